# baseline (speedup 1.0000x reference)
import functools

import jax
import jax.numpy as jnp
from jax import lax
from jax.experimental import pallas as pl
from jax.experimental.pallas import tpu as pltpu
from jax.experimental.pallas import tpu_sc as plsc

V = 1000
VP = 1024
NTOK = 51200
NC, NS, L = 2, 16, 16
NW = NC * NS
PER_W = NTOK // NW
CH = 32
NCHUNK = PER_W // CH
C8 = V // 8
RB = NTOK // 128
BLK = 128 * VP


def _logz_body(tab_ref, out_ref):
    x = tab_ref[...]
    m = jnp.max(x, axis=1, keepdims=True)
    e = jnp.exp(x - m)
    out_ref[...] = jnp.log(jnp.sum(e, axis=1, keepdims=True)) + m


_logz_call = pl.pallas_call(
    _logz_body,
    out_shape=jax.ShapeDtypeStruct((V, 1), jnp.float32),
)


def _loss_body(p_ref, out_ref):
    s = jnp.sum(p_ref[...]) * jnp.float32(1.0 / NTOK)
    out_ref[...] = jnp.full((1, 1), s, jnp.float32)


_loss_call = pl.pallas_call(
    _loss_body,
    out_shape=jax.ShapeDtypeStruct((1, 1), jnp.float32),
)


def _relayout_body(in_ref, out_ref):
    x = in_ref[...]
    x4 = x.reshape(16, 8, 8, 128)
    y = jnp.swapaxes(x4, 1, 2)
    for j in range(8):
        z = y[:, j].reshape(128, 128)
        zt = z.T
        n = 13 if j == 7 else 16
        out_ref[pl.ds(16 * j, n), 0] = zt.reshape(16, 8, 128)[:n]


_relayout_call = pl.pallas_call(
    _relayout_body,
    grid=(RB,),
    in_specs=[pl.BlockSpec((BLK,), lambda rb: (rb,))],
    out_specs=pl.BlockSpec((C8, 1, 8, 128), lambda rb: (0, rb, 0, 0)),
    out_shape=jax.ShapeDtypeStruct((C8, RB, 8, 128), jnp.float32),
    compiler_params=pltpu.CompilerParams(
        dimension_semantics=("parallel",)),
)


@functools.cache
def _make_sc_gather():
    mesh = plsc.VectorSubcoreMesh(core_axis_name="c", subcore_axis_name="s")
    return pl.kernel(
        _sc_gather_body,
        mesh=mesh,
        compiler_params=pltpu.CompilerParams(
            use_tc_tiling_on_sc=False, needs_layout_passes=False),
        out_type=[
            jax.ShapeDtypeStruct((NTOK, VP), jnp.float32),
            jax.ShapeDtypeStruct((NW, L), jnp.float32),
        ],
        scratch_types=[
            pltpu.VMEM((PER_W,), jnp.int32),
            pltpu.VMEM((PER_W,), jnp.int32),
            pltpu.VMEM((V,), jnp.float32),
            pltpu.VMEM((L,), jnp.float32),
            pltpu.VMEM((CH, V), jnp.float32),
            pltpu.VMEM((CH, V), jnp.float32),
            pltpu.SemaphoreType.DMA,
            pltpu.SemaphoreType.DMA,
            pltpu.SemaphoreType.DMA,
            pltpu.SemaphoreType.DMA,
        ],
    )


def _sc_gather_body(table_hbm, idx_hbm, tgt_hbm, logz_hbm, out_hbm, part_hbm,
                    idx_v, tgt_v, logz_v, acc_v, buf0, buf1,
                    gs0, gs1, ss0, ss1):
    wid = lax.axis_index("s") * NC + lax.axis_index("c")
    base = wid * PER_W
    bufs = (buf0, buf1)
    gsems = (gs0, gs1)
    ssems = (ss0, ss1)

    pltpu.sync_copy(idx_hbm.at[pl.ds(base, PER_W)], idx_v)
    pltpu.sync_copy(tgt_hbm.at[pl.ds(base, PER_W)], tgt_v)
    pltpu.sync_copy(logz_hbm, logz_v)
    acc_v[...] = jnp.zeros((L,), jnp.float32)

    def _gather(c, b):
        pltpu.make_async_copy(
            table_hbm.at[idx_v.at[pl.ds(c * CH, CH)]], bufs[b], gsems[b]
        ).start()

    def _gather_wait(c, b):
        pltpu.make_async_copy(
            table_hbm.at[idx_v.at[pl.ds(c * CH, CH)]], bufs[b], gsems[b]
        ).wait()

    def _scatter(c, b):
        pltpu.make_async_copy(
            bufs[b], out_hbm.at[pl.ds(base + c * CH, CH), pl.ds(0, V)],
            ssems[b]
        ).start()

    def _scatter_wait(c, b):
        pltpu.make_async_copy(
            bufs[b], out_hbm.at[pl.ds(base + c * CH, CH), pl.ds(0, V)],
            ssems[b]
        ).wait()

    iota = lax.iota(jnp.int32, L)

    def _loss_update(c, b):
        for g in range(CH // L):
            off = c * CH + g * L
            rows = iota + g * L
            cols = tgt_v[pl.ds(off, L)]
            ii = idx_v[pl.ds(off, L)]
            vv = plsc.load_gather(bufs[b], [rows, cols])
            zz = plsc.load_gather(logz_v, [ii])
            acc_v[...] = acc_v[...] + (zz - vv)

    _gather(0, 0)

    def _pair(i, _):
        for b in range(2):
            c = i * 2 + b
            nb = 1 - b

            @pl.when(c + 1 < NCHUNK)
            def _():
                @pl.when(c >= 1)
                def _():
                    _scatter_wait(c - 1, nb)
                _gather(c + 1, nb)

            _gather_wait(c, b)
            _loss_update(c, b)
            _scatter(c, b)
        return ()

    lax.fori_loop(0, NCHUNK // 2, _pair, ())

    _scatter_wait(NCHUNK - 2, 0)
    _scatter_wait(NCHUNK - 1, 1)
    pltpu.sync_copy(acc_v, part_hbm.at[wid])


def kernel(idx, targets, table):
    idx_f = idx.reshape(-1).astype(jnp.int32)
    tgt_f = targets.reshape(-1).astype(jnp.int32)
    logz = _logz_call(table).reshape(-1)
    raw, parts = _make_sc_gather()(table, idx_f, tgt_f, logz)
    logits_t = _relayout_call(raw.reshape(-1))
    logits = logits_t.transpose(1, 3, 0, 2).reshape(NTOK, V)
    loss = _loss_call(parts)[0, 0]
    return logits, loss


# speedup vs baseline: 1.4564x; 1.4564x over previous
import functools

import jax
import jax.numpy as jnp
from jax import lax
from jax.experimental import pallas as pl
from jax.experimental.pallas import tpu as pltpu
from jax.experimental.pallas import tpu_sc as plsc

V = 1000
VP = 1024
NTOK = 51200
NC, NS, L = 2, 16, 16
NW = NC * NS
PER_W = NTOK // NW
CH = 32
NCHUNK = PER_W // CH
C8 = V // 8
RB = NTOK // 128
BLK = 128 * VP


def _logz_body(tab_ref, out_ref):
    x = tab_ref[...]
    m = jnp.max(x, axis=1, keepdims=True)
    e = jnp.exp(x - m)
    out_ref[...] = jnp.log(jnp.sum(e, axis=1, keepdims=True)) + m


_logz_call = pl.pallas_call(
    _logz_body,
    out_shape=jax.ShapeDtypeStruct((V, 1), jnp.float32),
)


def _loss_body(p_ref, out_ref):
    s = jnp.sum(p_ref[...]) * jnp.float32(1.0 / NTOK)
    out_ref[...] = jnp.full((1, 1), s, jnp.float32)


_loss_call = pl.pallas_call(
    _loss_body,
    out_shape=jax.ShapeDtypeStruct((1, 1), jnp.float32),
)


RBM = 4


def _relayout_body(in_ref, out_ref):
    x = in_ref[...]
    for s in range(RBM):
        xs = x[s * BLK:(s + 1) * BLK]
        x4 = xs.reshape(16, 8, 8, 128)
        y = jnp.swapaxes(x4, 1, 2)
        for j in range(8):
            z = y[:, j].reshape(128, 128)
            zt = z.T
            n = 13 if j == 7 else 16
            out_ref[pl.ds(16 * j, n), s] = zt.reshape(16, 8, 128)[:n]


_relayout_call = pl.pallas_call(
    _relayout_body,
    grid=(RB // RBM,),
    in_specs=[pl.BlockSpec((RBM * BLK,), lambda rb: (rb,))],
    out_specs=pl.BlockSpec((C8, RBM, 8, 128), lambda rb: (0, rb, 0, 0)),
    out_shape=jax.ShapeDtypeStruct((C8, RB, 8, 128), jnp.float32),
    compiler_params=pltpu.CompilerParams(
        dimension_semantics=("parallel",)),
)


@functools.cache
def _make_sc_gather():
    mesh = plsc.VectorSubcoreMesh(core_axis_name="c", subcore_axis_name="s")
    return pl.kernel(
        _sc_gather_body,
        mesh=mesh,
        compiler_params=pltpu.CompilerParams(
            use_tc_tiling_on_sc=False, needs_layout_passes=False),
        out_type=[
            jax.ShapeDtypeStruct((NTOK, VP), jnp.float32),
            jax.ShapeDtypeStruct((NW, L), jnp.float32),
        ],
        scratch_types=[
            pltpu.VMEM((PER_W,), jnp.int32),
            pltpu.VMEM((PER_W,), jnp.int32),
            pltpu.VMEM((V,), jnp.float32),
            pltpu.VMEM((L,), jnp.float32),
            pltpu.VMEM((CH, V), jnp.float32),
            pltpu.VMEM((CH, V), jnp.float32),
            pltpu.SemaphoreType.DMA,
            pltpu.SemaphoreType.DMA,
            pltpu.SemaphoreType.DMA,
            pltpu.SemaphoreType.DMA,
        ],
    )


def _sc_gather_body(table_hbm, idx_hbm, tgt_hbm, logz_hbm, out_hbm, part_hbm,
                    idx_v, tgt_v, logz_v, acc_v, buf0, buf1,
                    gs0, gs1, ss0, ss1):
    wid = lax.axis_index("s") * NC + lax.axis_index("c")
    base = wid * PER_W
    bufs = (buf0, buf1)
    gsems = (gs0, gs1)
    ssems = (ss0, ss1)

    pltpu.sync_copy(idx_hbm.at[pl.ds(base, PER_W)], idx_v)
    pltpu.sync_copy(tgt_hbm.at[pl.ds(base, PER_W)], tgt_v)
    pltpu.sync_copy(logz_hbm, logz_v)
    acc_v[...] = jnp.zeros((L,), jnp.float32)

    def _gather(c, b):
        pltpu.make_async_copy(
            table_hbm.at[idx_v.at[pl.ds(c * CH, CH)]], bufs[b], gsems[b]
        ).start()

    def _gather_wait(c, b):
        pltpu.make_async_copy(
            table_hbm.at[idx_v.at[pl.ds(c * CH, CH)]], bufs[b], gsems[b]
        ).wait()

    def _scatter(c, b):
        pltpu.make_async_copy(
            bufs[b], out_hbm.at[pl.ds(base + c * CH, CH), pl.ds(0, V)],
            ssems[b]
        ).start()

    def _scatter_wait(c, b):
        pltpu.make_async_copy(
            bufs[b], out_hbm.at[pl.ds(base + c * CH, CH), pl.ds(0, V)],
            ssems[b]
        ).wait()

    iota = lax.iota(jnp.int32, L)

    def _loss_update(c, b):
        for g in range(CH // L):
            off = c * CH + g * L
            rows = iota + g * L
            cols = tgt_v[pl.ds(off, L)]
            ii = idx_v[pl.ds(off, L)]
            vv = plsc.load_gather(bufs[b], [rows, cols])
            zz = plsc.load_gather(logz_v, [ii])
            acc_v[...] = acc_v[...] + (zz - vv)

    _gather(0, 0)

    def _pair(i, _):
        for b in range(2):
            c = i * 2 + b
            nb = 1 - b

            @pl.when(c + 1 < NCHUNK)
            def _():
                @pl.when(c >= 1)
                def _():
                    _scatter_wait(c - 1, nb)
                _gather(c + 1, nb)

            _gather_wait(c, b)
            _loss_update(c, b)
            _scatter(c, b)
        return ()

    lax.fori_loop(0, NCHUNK // 2, _pair, ())

    _scatter_wait(NCHUNK - 2, 0)
    _scatter_wait(NCHUNK - 1, 1)
    pltpu.sync_copy(acc_v, part_hbm.at[wid])


def kernel(idx, targets, table):
    idx_f = idx.reshape(-1).astype(jnp.int32)
    tgt_f = targets.reshape(-1).astype(jnp.int32)
    logz = _logz_call(table).reshape(-1)
    raw, parts = _make_sc_gather()(table, idx_f, tgt_f, logz)
    logits_t = _relayout_call(raw.reshape(-1))
    logits = logits_t.transpose(1, 3, 0, 2).reshape(NTOK, V)
    loss = _loss_call(parts)[0, 0]
    return logits, loss


# speedup vs baseline: 1.5705x; 1.0784x over previous
import functools

import jax
import jax.numpy as jnp
from jax import lax
from jax.experimental import pallas as pl
from jax.experimental.pallas import tpu as pltpu
from jax.experimental.pallas import tpu_sc as plsc

V = 1000
VP = 1024
NTOK = 51200
NC, NS, L = 2, 16, 16
NW = NC * NS
PER_W = NTOK // NW
CH = 32
NCHUNK = PER_W // CH
C8 = V // 8
RB = NTOK // 128
BLK = 128 * VP


def _logz_body(tab_ref, out_ref):
    x = tab_ref[...]
    m = jnp.max(x, axis=1, keepdims=True)
    e = jnp.exp(x - m)
    out_ref[...] = jnp.log(jnp.sum(e, axis=1, keepdims=True)) + m


_logz_call = pl.pallas_call(
    _logz_body,
    out_shape=jax.ShapeDtypeStruct((V, 1), jnp.float32),
)


def _loss_body(p_ref, out_ref):
    s = jnp.sum(p_ref[...]) * jnp.float32(1.0 / NTOK)
    out_ref[...] = jnp.full((1, 1), s, jnp.float32)


_loss_call = pl.pallas_call(
    _loss_body,
    out_shape=jax.ShapeDtypeStruct((1, 1), jnp.float32),
)


RBM = 8


def _relayout_body(in_ref, out_ref):
    x = in_ref[...]
    for s in range(RBM):
        xs = x[s * BLK:(s + 1) * BLK]
        x4 = xs.reshape(16, 8, 8, 128)
        y = jnp.swapaxes(x4, 1, 2)
        for j in range(8):
            z = y[:, j].reshape(128, 128)
            zt = z.T
            n = 13 if j == 7 else 16
            out_ref[pl.ds(16 * j, n), s] = zt.reshape(16, 8, 128)[:n]


_relayout_call = pl.pallas_call(
    _relayout_body,
    grid=(RB // RBM,),
    in_specs=[pl.BlockSpec((RBM * BLK,), lambda rb: (rb,))],
    out_specs=pl.BlockSpec((C8, RBM, 8, 128), lambda rb: (0, rb, 0, 0)),
    out_shape=jax.ShapeDtypeStruct((C8, RB, 8, 128), jnp.float32),
    compiler_params=pltpu.CompilerParams(
        dimension_semantics=("parallel",)),
)


@functools.cache
def _make_sc_gather():
    mesh = plsc.VectorSubcoreMesh(core_axis_name="c", subcore_axis_name="s")
    return pl.kernel(
        _sc_gather_body,
        mesh=mesh,
        compiler_params=pltpu.CompilerParams(
            use_tc_tiling_on_sc=False, needs_layout_passes=False),
        out_type=[
            jax.ShapeDtypeStruct((NTOK, VP), jnp.float32),
            jax.ShapeDtypeStruct((NW, L), jnp.float32),
        ],
        scratch_types=[
            pltpu.VMEM((PER_W,), jnp.int32),
            pltpu.VMEM((PER_W,), jnp.int32),
            pltpu.VMEM((V,), jnp.float32),
            pltpu.VMEM((L,), jnp.float32),
            pltpu.VMEM((CH, V), jnp.float32),
            pltpu.VMEM((CH, V), jnp.float32),
            pltpu.SemaphoreType.DMA,
            pltpu.SemaphoreType.DMA,
            pltpu.SemaphoreType.DMA,
            pltpu.SemaphoreType.DMA,
        ],
    )


def _sc_gather_body(table_hbm, idx_hbm, tgt_hbm, logz_hbm, out_hbm, part_hbm,
                    idx_v, tgt_v, logz_v, acc_v, buf0, buf1,
                    gs0, gs1, ss0, ss1):
    wid = lax.axis_index("s") * NC + lax.axis_index("c")
    base = wid * PER_W
    bufs = (buf0, buf1)
    gsems = (gs0, gs1)
    ssems = (ss0, ss1)

    pltpu.sync_copy(idx_hbm.at[pl.ds(base, PER_W)], idx_v)
    pltpu.sync_copy(tgt_hbm.at[pl.ds(base, PER_W)], tgt_v)
    pltpu.sync_copy(logz_hbm, logz_v)
    acc_v[...] = jnp.zeros((L,), jnp.float32)

    def _gather(c, b):
        pltpu.make_async_copy(
            table_hbm.at[idx_v.at[pl.ds(c * CH, CH)]], bufs[b], gsems[b]
        ).start()

    def _gather_wait(c, b):
        pltpu.make_async_copy(
            table_hbm.at[idx_v.at[pl.ds(c * CH, CH)]], bufs[b], gsems[b]
        ).wait()

    def _scatter(c, b):
        pltpu.make_async_copy(
            bufs[b], out_hbm.at[pl.ds(base + c * CH, CH), pl.ds(0, V)],
            ssems[b]
        ).start()

    def _scatter_wait(c, b):
        pltpu.make_async_copy(
            bufs[b], out_hbm.at[pl.ds(base + c * CH, CH), pl.ds(0, V)],
            ssems[b]
        ).wait()

    iota = lax.iota(jnp.int32, L)

    def _loss_update(c, b):
        for g in range(CH // L):
            off = c * CH + g * L
            rows = iota + g * L
            cols = tgt_v[pl.ds(off, L)]
            ii = idx_v[pl.ds(off, L)]
            vv = plsc.load_gather(bufs[b], [rows, cols])
            zz = plsc.load_gather(logz_v, [ii])
            acc_v[...] = acc_v[...] + (zz - vv)

    _gather(0, 0)

    def _pair(i, _):
        for b in range(2):
            c = i * 2 + b
            nb = 1 - b

            @pl.when(c + 1 < NCHUNK)
            def _():
                @pl.when(c >= 1)
                def _():
                    _scatter_wait(c - 1, nb)
                _gather(c + 1, nb)

            _gather_wait(c, b)
            _loss_update(c, b)
            _scatter(c, b)
        return ()

    lax.fori_loop(0, NCHUNK // 2, _pair, ())

    _scatter_wait(NCHUNK - 2, 0)
    _scatter_wait(NCHUNK - 1, 1)
    pltpu.sync_copy(acc_v, part_hbm.at[wid])


def kernel(idx, targets, table):
    idx_f = idx.reshape(-1).astype(jnp.int32)
    tgt_f = targets.reshape(-1).astype(jnp.int32)
    logz = _logz_call(table).reshape(-1)
    raw, parts = _make_sc_gather()(table, idx_f, tgt_f, logz)
    logits_t = _relayout_call(raw.reshape(-1))
    logits = logits_t.transpose(1, 3, 0, 2).reshape(NTOK, V)
    loss = _loss_call(parts)[0, 0]
    return logits, loss
